# baseline (speedup 1.0000x reference)
import functools

import jax
import jax.numpy as jnp
from jax.experimental import pallas as pl
from jax.experimental.pallas import tpu as pltpu

_BT = 1024
_E = 8
_C = 2


def _moe_body(x_ref, w1_ref, w2_ref, b_ref, out_ref, *, E, C):
    x = x_ref[...]
    xh = x.astype(jnp.bfloat16)
    xl = (x - xh.astype(jnp.float32)).astype(jnp.bfloat16)
    y1 = jax.lax.dot_general(
        xh, w1_ref[...], (((1,), (0,)), ((), ())),
        preferred_element_type=jnp.float32,
    )
    y2 = jax.lax.dot_general(
        xl, w2_ref[...], (((1,), (0,)), ((), ())),
        preferred_element_type=jnp.float32,
    )
    gate = y1[:, 0:E] + y1[:, E:2 * E] + y2 + b_ref[:, 0:E]
    eo = y1[:, 2 * E:2 * E + C * E] + b_ref[:, E:E + C * E]
    bt = gate.shape[0]
    ids = jax.lax.broadcasted_iota(jnp.int32, (bt, E), 1)
    m1 = jnp.max(gate, axis=1, keepdims=True)
    idx1 = jnp.min(jnp.where(gate == m1, ids, E), axis=1, keepdims=True)
    g2 = jnp.where(ids == idx1, -jnp.inf, gate)
    m2 = jnp.max(g2, axis=1, keepdims=True)
    idx2 = jnp.min(jnp.where(g2 == m2, ids, E), axis=1, keepdims=True)
    w1 = 1.0 / (1.0 + jnp.exp(m2 - m1))
    w2 = 1.0 - w1
    wts = jnp.where(ids == idx1, w1, 0.0) + jnp.where(ids == idx2, w2, 0.0)
    outs = [
        jnp.sum(wts * eo[:, c * E:(c + 1) * E], axis=1, keepdims=True)
        for c in range(C)
    ]
    out_ref[...] = jnp.concatenate(outs, axis=1)


def kernel(hidden_states, gate_w, gate_b, expert_w, expert_b):
    T, D = hidden_states.shape
    E = gate_w.shape[1]
    C = expert_w.shape[2]
    we = jnp.transpose(expert_w, (1, 2, 0)).reshape(D, C * E)
    gh = gate_w.astype(jnp.bfloat16)
    gl = (gate_w - gh.astype(jnp.float32)).astype(jnp.bfloat16)
    w1 = jnp.concatenate([gh, gl, we.astype(jnp.bfloat16)], axis=1)
    b = jnp.concatenate(
        [gate_b.reshape(1, E), jnp.transpose(expert_b, (1, 0)).reshape(1, C * E)],
        axis=1,
    )
    n1 = 2 * E + C * E
    return pl.pallas_call(
        functools.partial(_moe_body, E=E, C=C),
        grid=(T // _BT,),
        in_specs=[
            pl.BlockSpec((_BT, D), lambda i: (i, 0)),
            pl.BlockSpec((D, n1), lambda i: (0, 0)),
            pl.BlockSpec((D, E), lambda i: (0, 0)),
            pl.BlockSpec((1, E + C * E), lambda i: (0, 0)),
        ],
        out_specs=pl.BlockSpec((_BT, 2), lambda i: (i, 0)),
        out_shape=jax.ShapeDtypeStruct((T, 2), jnp.float32),
        compiler_params=pltpu.CompilerParams(
            dimension_semantics=("arbitrary",),
        ),
    )(hidden_states, w1, gh, b)


# speedup vs baseline: 1.3539x; 1.3539x over previous
import functools

import jax
import jax.numpy as jnp
from jax.experimental import pallas as pl
from jax.experimental.pallas import tpu as pltpu

_BT = 1024
_E = 8
_C = 2
_NCOLS = _E + _E * _C


def _moe_body(x_ref, w_ref, b_ref, out_ref, *, E, C):
    x = x_ref[...]
    xh = x.astype(jnp.bfloat16)
    y = (
        jax.lax.dot_general(
            xh, w_ref[...], (((1,), (0,)), ((), ())),
            preferred_element_type=jnp.float32,
        )
        + b_ref[...]
    )
    gate = y[:, 0:E]
    bt = gate.shape[0]
    ids = jax.lax.broadcasted_iota(jnp.int32, (bt, E), 1)
    m1 = jnp.max(gate, axis=1, keepdims=True)
    idx1 = jnp.min(jnp.where(gate == m1, ids, E), axis=1, keepdims=True)
    g2 = jnp.where(ids == idx1, -jnp.inf, gate)
    m2 = jnp.max(g2, axis=1, keepdims=True)
    idx2 = jnp.min(jnp.where(g2 == m2, ids, E), axis=1, keepdims=True)
    w1 = 1.0 / (1.0 + jnp.exp(m2 - m1))
    w2 = 1.0 - w1
    wts = jnp.where(ids == idx1, w1, 0.0) + jnp.where(ids == idx2, w2, 0.0)
    outs = [
        jnp.sum(wts * y[:, (1 + c) * E:(2 + c) * E], axis=1, keepdims=True)
        for c in range(C)
    ]
    out_ref[...] = jnp.concatenate(outs, axis=1)


def kernel(hidden_states, gate_w, gate_b, expert_w, expert_b):
    T, D = hidden_states.shape
    E = gate_w.shape[1]
    C = expert_w.shape[2]
    we = jnp.transpose(expert_w, (1, 2, 0)).reshape(D, C * E)
    w = jnp.concatenate([gate_w, we], axis=1).astype(jnp.bfloat16)
    b = jnp.concatenate(
        [gate_b.reshape(1, E), jnp.transpose(expert_b, (1, 0)).reshape(1, C * E)],
        axis=1,
    )
    return pl.pallas_call(
        functools.partial(_moe_body, E=E, C=C),
        grid=(T // _BT,),
        in_specs=[
            pl.BlockSpec((_BT, D), lambda i: (i, 0)),
            pl.BlockSpec((D, _NCOLS), lambda i: (0, 0)),
            pl.BlockSpec((1, _NCOLS), lambda i: (0, 0)),
        ],
        out_specs=pl.BlockSpec((_BT, 2), lambda i: (i, 0)),
        out_shape=jax.ShapeDtypeStruct((T, 2), jnp.float32),
        compiler_params=pltpu.CompilerParams(
            dimension_semantics=("arbitrary",),
        ),
    )(hidden_states, w, b)
